# baseline (speedup 1.0000x reference)
import functools

import jax
import jax.numpy as jnp
from jax import lax
from jax.experimental import pallas as pl
from jax.experimental.pallas import tpu as pltpu
from jax.experimental.pallas import tpu_sc as plsc

_TIME_STEPS = 1000
_EMBED_DIM = 128
_BATCH = 16384


def _make_sc_gather(batch, dim, vocab, chunk=64, hbm_chunks=2):
    info = plsc.get_sparse_core_info()
    nc, ns = info.num_cores, info.num_subcores
    nw = nc * ns
    assert batch % (8 * nw) == 0
    b_per_w = batch // nw
    assert b_per_w % chunk == 0
    n_chunks = b_per_w // chunk
    stage_tiles = 5
    rows_per_stager = vocab // stage_tiles
    assert vocab % stage_tiles == 0 and rows_per_stager % 8 == 0

    mesh = plsc.VectorSubcoreMesh(core_axis_name="c", subcore_axis_name="s")

    @functools.partial(
        pl.kernel,
        mesh=mesh,
        out_type=jax.ShapeDtypeStruct((batch, dim), jnp.float32),
        scratch_types=[
            pltpu.VMEM((b_per_w,), jnp.int32),
            pltpu.VMEM((b_per_w, dim), jnp.float32),
            pltpu.VMEM_SHARED((vocab, dim), jnp.float32),
        ]
        + [pltpu.SemaphoreType.DMA] * (n_chunks + 2),
    )
    def emb_kernel(idx_hbm, table_hbm, out_hbm, idx_v, rows_v, table_sp, *sems):
        tsem = sems[0]
        gsems = sems[1 : 1 + n_chunks]
        ssem = sems[1 + n_chunks]
        sid = lax.axis_index("s")
        wid = sid * nc + lax.axis_index("c")
        base = wid * b_per_w

        @pl.when(sid < stage_tiles)
        def _():
            pltpu.async_copy(
                table_hbm.at[pl.ds(sid * rows_per_stager, rows_per_stager)],
                table_sp.at[pl.ds(sid * rows_per_stager, rows_per_stager)],
                tsem,
            )

        pltpu.sync_copy(idx_hbm.at[pl.ds(base, b_per_w)], idx_v)

        gathers = [None] * n_chunks
        stores = []
        for c in range(hbm_chunks):
            gathers[c] = pltpu.async_copy(
                table_hbm.at[idx_v.at[pl.ds(c * chunk, chunk)]],
                rows_v.at[pl.ds(c * chunk, chunk)],
                gsems[c],
            )
        for c in range(hbm_chunks):
            gathers[c].wait()
            stores.append(
                pltpu.async_copy(
                    rows_v.at[pl.ds(c * chunk, chunk)],
                    out_hbm.at[pl.ds(base + c * chunk, chunk)],
                    ssem,
                )
            )

        @pl.when(sid < stage_tiles)
        def _():
            pltpu.make_async_copy(
                table_hbm.at[pl.ds(sid * rows_per_stager, rows_per_stager)],
                table_sp.at[pl.ds(sid * rows_per_stager, rows_per_stager)],
                tsem,
            ).wait()

        plsc.subcore_barrier()
        for c in range(hbm_chunks, n_chunks):
            gathers[c] = pltpu.async_copy(
                table_sp.at[idx_v.at[pl.ds(c * chunk, chunk)]],
                rows_v.at[pl.ds(c * chunk, chunk)],
                gsems[c],
            )
        for c in range(hbm_chunks, n_chunks):
            gathers[c].wait()
            stores.append(
                pltpu.async_copy(
                    rows_v.at[pl.ds(c * chunk, chunk)],
                    out_hbm.at[pl.ds(base + c * chunk, chunk)],
                    ssem,
                )
            )
        for s in stores:
            s.wait()

    return emb_kernel


def kernel(x, table):
    emb = _make_sc_gather(_BATCH, _EMBED_DIM, _TIME_STEPS)
    return emb(x.astype(jnp.int32), table)


# speedup vs baseline: 1.0359x; 1.0359x over previous
import functools

import jax
import jax.numpy as jnp
from jax import lax
from jax.experimental import pallas as pl
from jax.experimental.pallas import tpu as pltpu
from jax.experimental.pallas import tpu_sc as plsc

_TIME_STEPS = 1000
_EMBED_DIM = 128
_BATCH = 16384


def _make_sc_gather(batch, dim, vocab, chunk=64):
    info = plsc.get_sparse_core_info()
    nc, ns = info.num_cores, info.num_subcores
    nw = nc * ns
    assert batch % (8 * nw) == 0
    b_per_w = batch // nw
    assert b_per_w % chunk == 0
    n_chunks = b_per_w // chunk
    stage_tiles = 5
    rows_per_stager = vocab // stage_tiles
    assert vocab % stage_tiles == 0 and rows_per_stager % 8 == 0

    mesh = plsc.VectorSubcoreMesh(core_axis_name="c", subcore_axis_name="s")

    @functools.partial(
        pl.kernel,
        mesh=mesh,
        out_type=jax.ShapeDtypeStruct((batch, dim), jnp.float32),
        scratch_types=[
            pltpu.VMEM((b_per_w,), jnp.int32),
            pltpu.VMEM((b_per_w, dim), jnp.float32),
            pltpu.VMEM_SHARED((vocab, dim), jnp.float32),
        ]
        + [pltpu.SemaphoreType.DMA] * (n_chunks + 2),
    )
    def emb_kernel(idx_hbm, table_hbm, out_hbm, idx_v, rows_v, table_sp, *sems):
        tsem = sems[0]
        gsems = sems[1 : 1 + n_chunks]
        ssem = sems[1 + n_chunks]
        sid = lax.axis_index("s")
        wid = sid * nc + lax.axis_index("c")
        base = wid * b_per_w

        @pl.when(sid < stage_tiles)
        def _():
            pltpu.async_copy(
                table_hbm.at[pl.ds(sid * rows_per_stager, rows_per_stager)],
                table_sp.at[pl.ds(sid * rows_per_stager, rows_per_stager)],
                tsem,
            )

        pltpu.sync_copy(idx_hbm.at[pl.ds(base, b_per_w)], idx_v)

        @pl.when(sid < stage_tiles)
        def _():
            pltpu.make_async_copy(
                table_hbm.at[pl.ds(sid * rows_per_stager, rows_per_stager)],
                table_sp.at[pl.ds(sid * rows_per_stager, rows_per_stager)],
                tsem,
            ).wait()

        plsc.subcore_barrier()
        gathers = []
        for c in range(n_chunks):
            gathers.append(
                pltpu.async_copy(
                    table_sp.at[idx_v.at[pl.ds(c * chunk, chunk)]],
                    rows_v.at[pl.ds(c * chunk, chunk)],
                    gsems[c],
                )
            )
        stores = []
        for c in range(n_chunks):
            gathers[c].wait()
            stores.append(
                pltpu.async_copy(
                    rows_v.at[pl.ds(c * chunk, chunk)],
                    out_hbm.at[pl.ds(base + c * chunk, chunk)],
                    ssem,
                )
            )
        for s in stores:
            s.wait()

    return emb_kernel


def kernel(x, table):
    emb = _make_sc_gather(_BATCH, _EMBED_DIM, _TIME_STEPS)
    return emb(x.astype(jnp.int32), table)
